# baseline (speedup 1.0000x reference)
import jax
import jax.numpy as jnp
from jax.experimental import pallas as pl
from jax.experimental.pallas import tpu as pltpu


def _proj_prep_kernel(p_ref, pbf_ref, g_ref):
    p = p_ref[...].astype(jnp.bfloat16)
    pbf_ref[...] = p
    g_ref[...] = jax.lax.dot_general(
        p, p, (((0,), (0,)), ((), ())),
        preferred_element_type=jnp.float32).astype(jnp.bfloat16)


def _main_kernel(x_ref, pbf_ref, g_ref, c_ref, xhv_ref, m_ref, f_ref):
    xf = x_ref[...]
    xb = xf.astype(jnp.bfloat16)
    xhv_ref[...] = jax.lax.dot_general(
        xb, pbf_ref[...], (((1,), (1,)), ((), ())),
        preferred_element_type=jnp.float32)
    t = jax.lax.dot_general(
        xb, g_ref[...], (((1,), (1,)), ((), ())),
        preferred_element_type=jnp.float32)
    s = jnp.sum(t * xf, axis=1, keepdims=True)
    f_ref[...] = 1.0 / (jnp.sqrt(s) + 1e-12)
    c = c_ref[...]
    cs = jnp.sum(c * c, axis=1, keepdims=True)
    cn = (c * (1.0 / (jnp.sqrt(cs) + 1e-12))).astype(jnp.bfloat16)
    m_ref[...] = jax.lax.dot_general(
        cn, pbf_ref[...], (((1,), (0,)), ((), ())),
        preferred_element_type=jnp.float32).astype(jnp.bfloat16)


def _preds_kernel(x_ref, m_ref, f_ref, preds_ref):
    p = jax.lax.dot_general(
        x_ref[...].astype(jnp.bfloat16), m_ref[...], (((1,), (1,)), ((), ())),
        preferred_element_type=jnp.float32)
    preds_ref[...] = p * f_ref[...]


@jax.jit
def kernel(x, projection, centroids):
    B, F = x.shape
    D, _ = projection.shape
    C, _ = centroids.shape

    pbf, g = pl.pallas_call(
        _proj_prep_kernel,
        in_specs=[pl.BlockSpec((D, F), lambda: (0, 0))],
        out_specs=[
            pl.BlockSpec((D, F), lambda: (0, 0)),
            pl.BlockSpec((F, F), lambda: (0, 0)),
        ],
        out_shape=[
            jax.ShapeDtypeStruct((D, F), jnp.bfloat16),
            jax.ShapeDtypeStruct((F, F), jnp.bfloat16),
        ],
    )(projection)

    BI = 512
    NS = B // BI
    BCC = C // NS
    xhv, m, f = pl.pallas_call(
        _main_kernel,
        grid=(NS,),
        in_specs=[
            pl.BlockSpec((BI, F), lambda i: (i, 0)),
            pl.BlockSpec((D, F), lambda i: (0, 0)),
            pl.BlockSpec((F, F), lambda i: (0, 0)),
            pl.BlockSpec((BCC, D), lambda i: (i, 0)),
        ],
        out_specs=[
            pl.BlockSpec((BI, D), lambda i: (i, 0)),
            pl.BlockSpec((BCC, F), lambda i: (i, 0)),
            pl.BlockSpec((BI, 1), lambda i: (i, 0)),
        ],
        out_shape=[
            jax.ShapeDtypeStruct((B, D), jnp.float32),
            jax.ShapeDtypeStruct((C, F), jnp.bfloat16),
            jax.ShapeDtypeStruct((B, 1), jnp.float32),
        ],
        compiler_params=pltpu.CompilerParams(
            dimension_semantics=("arbitrary",),
        ),
    )(x, pbf, g, centroids)

    BP = 1024
    preds = pl.pallas_call(
        _preds_kernel,
        grid=(B // BP,),
        in_specs=[
            pl.BlockSpec((BP, F), lambda i: (i, 0)),
            pl.BlockSpec((C, F), lambda i: (0, 0)),
            pl.BlockSpec((BP, 1), lambda i: (i, 0)),
        ],
        out_specs=pl.BlockSpec((BP, C), lambda i: (i, 0)),
        out_shape=jax.ShapeDtypeStruct((B, C), jnp.float32),
        compiler_params=pltpu.CompilerParams(
            dimension_semantics=("arbitrary",),
        ),
    )(x, m, f)
    return (preds, xhv)


# speedup vs baseline: 1.0159x; 1.0159x over previous
import jax
import jax.numpy as jnp
from jax.experimental import pallas as pl
from jax.experimental.pallas import tpu as pltpu


def _main_kernel(x_ref, p_ref, c_ref, xhv_ref, m_ref, f_ref, pbf_s, g_s):
    @pl.when(pl.program_id(0) == 0)
    def _prep():
        pb = p_ref[...].astype(jnp.bfloat16)
        pbf_s[...] = pb
        g_s[...] = jax.lax.dot_general(
            pb, pb, (((0,), (0,)), ((), ())),
            preferred_element_type=jnp.float32).astype(jnp.bfloat16)

    xf = x_ref[...]
    xb = xf.astype(jnp.bfloat16)
    xhv_ref[...] = jax.lax.dot_general(
        xb, pbf_s[...], (((1,), (1,)), ((), ())),
        preferred_element_type=jnp.float32)
    t = jax.lax.dot_general(
        xb, g_s[...], (((1,), (1,)), ((), ())),
        preferred_element_type=jnp.float32)
    s = jnp.sum(t * xf, axis=1, keepdims=True)
    f_ref[...] = 1.0 / (jnp.sqrt(s) + 1e-12)
    c = c_ref[...]
    cs = jnp.sum(c * c, axis=1, keepdims=True)
    cn = (c * (1.0 / (jnp.sqrt(cs) + 1e-12))).astype(jnp.bfloat16)
    m_ref[...] = jax.lax.dot_general(
        cn, pbf_s[...], (((1,), (0,)), ((), ())),
        preferred_element_type=jnp.float32).astype(jnp.bfloat16)


def _preds_kernel(x_ref, m_ref, f_ref, preds_ref):
    p = jax.lax.dot_general(
        x_ref[...].astype(jnp.bfloat16), m_ref[...], (((1,), (1,)), ((), ())),
        preferred_element_type=jnp.float32)
    preds_ref[...] = p * f_ref[...]


@jax.jit
def kernel(x, projection, centroids):
    B, F = x.shape
    D, _ = projection.shape
    C, _ = centroids.shape

    BI = 256
    NS = B // BI
    BCC = C // NS
    xhv, m, f = pl.pallas_call(
        _main_kernel,
        grid=(NS,),
        in_specs=[
            pl.BlockSpec((BI, F), lambda i: (i, 0)),
            pl.BlockSpec((D, F), lambda i: (0, 0)),
            pl.BlockSpec((BCC, D), lambda i: (i, 0)),
        ],
        out_specs=[
            pl.BlockSpec((BI, D), lambda i: (i, 0)),
            pl.BlockSpec((BCC, F), lambda i: (i, 0)),
            pl.BlockSpec((BI, 1), lambda i: (i, 0)),
        ],
        out_shape=[
            jax.ShapeDtypeStruct((B, D), jnp.float32),
            jax.ShapeDtypeStruct((C, F), jnp.bfloat16),
            jax.ShapeDtypeStruct((B, 1), jnp.float32),
        ],
        scratch_shapes=[
            pltpu.VMEM((D, F), jnp.bfloat16),
            pltpu.VMEM((F, F), jnp.bfloat16),
        ],
        compiler_params=pltpu.CompilerParams(
            dimension_semantics=("arbitrary",),
        ),
    )(x, projection, centroids)

    BP = 1024
    preds = pl.pallas_call(
        _preds_kernel,
        grid=(B // BP,),
        in_specs=[
            pl.BlockSpec((BP, F), lambda i: (i, 0)),
            pl.BlockSpec((C, F), lambda i: (0, 0)),
            pl.BlockSpec((BP, 1), lambda i: (i, 0)),
        ],
        out_specs=pl.BlockSpec((BP, C), lambda i: (i, 0)),
        out_shape=jax.ShapeDtypeStruct((B, C), jnp.float32),
        compiler_params=pltpu.CompilerParams(
            dimension_semantics=("arbitrary",),
        ),
    )(x, m, f)
    return (preds, xhv)


# speedup vs baseline: 1.0400x; 1.0237x over previous
import jax
import jax.numpy as jnp
from jax.experimental import pallas as pl
from jax.experimental.pallas import tpu as pltpu

_BI = 256
_BCC = 128


def _fused_kernel(x_ref, p_ref, c_ref, xhv_ref, preds_ref,
                  xb_s, pbf_s, fac_s):
    i = pl.program_id(0)

    @pl.when(i == 0)
    def _prep():
        pb = p_ref[...].astype(jnp.bfloat16)
        pbf_s[...] = pb
        g = jax.lax.dot_general(
            pb, pb, (((0,), (0,)), ((), ())),
            preferred_element_type=jnp.float32).astype(jnp.bfloat16)
        xf = x_ref[...]
        xb = xf.astype(jnp.bfloat16)
        xb_s[...] = xb
        t = jax.lax.dot_general(
            xb, g, (((1,), (1,)), ((), ())),
            preferred_element_type=jnp.float32)
        s = jnp.sum(t * xf, axis=1, keepdims=True)
        fac_s[...] = 1.0 / (jnp.sqrt(s) + 1e-12)

    xb_i = xb_s[pl.ds(i * _BI, _BI), :]
    xhv_ref[...] = jax.lax.dot_general(
        xb_i, pbf_s[...], (((1,), (1,)), ((), ())),
        preferred_element_type=jnp.float32)

    @pl.when(i < pl.num_programs(0) // 2)
    def _centroid_step():
        c = c_ref[...]
        cs = jnp.sum(c * c, axis=1, keepdims=True)
        cn = (c * (1.0 / (jnp.sqrt(cs) + 1e-12))).astype(jnp.bfloat16)
        mb = jax.lax.dot_general(
            cn, pbf_s[...], (((1,), (0,)), ((), ())),
            preferred_element_type=jnp.float32).astype(jnp.bfloat16)
        pc = jax.lax.dot_general(
            xb_s[...], mb, (((1,), (1,)), ((), ())),
            preferred_element_type=jnp.float32)
        preds_ref[...] = pc * fac_s[...]


@jax.jit
def kernel(x, projection, centroids):
    B, F = x.shape
    D, _ = projection.shape
    C, _ = centroids.shape
    NS = B // _BI

    xhv, preds = pl.pallas_call(
        _fused_kernel,
        grid=(NS,),
        in_specs=[
            pl.BlockSpec((B, F), lambda i: (0, 0)),
            pl.BlockSpec((D, F), lambda i: (0, 0)),
            pl.BlockSpec((_BCC, D), lambda i: (jnp.minimum(i, 7), 0)),
        ],
        out_specs=[
            pl.BlockSpec((_BI, D), lambda i: (i, 0)),
            pl.BlockSpec((B, _BCC), lambda i: (0, jnp.minimum(i, 7))),
        ],
        out_shape=[
            jax.ShapeDtypeStruct((B, D), jnp.float32),
            jax.ShapeDtypeStruct((B, C), jnp.float32),
        ],
        scratch_shapes=[
            pltpu.VMEM((B, F), jnp.bfloat16),
            pltpu.VMEM((D, F), jnp.bfloat16),
            pltpu.VMEM((B, 1), jnp.float32),
        ],
        compiler_params=pltpu.CompilerParams(
            dimension_semantics=("arbitrary",),
        ),
    )(x, projection, centroids)
    return (preds, xhv)


# speedup vs baseline: 1.0704x; 1.0293x over previous
import jax
import jax.numpy as jnp
from jax.experimental import pallas as pl
from jax.experimental.pallas import tpu as pltpu

_BI = 256
_BCC = 128


def _fused_kernel(x_ref, p_ref, c_ref, xhv_ref, preds_ref,
                  xb_s, pbf_s, g_s, fac_s):
    i = pl.program_id(0)
    half = pl.num_programs(0) // 2

    @pl.when(i == 0)
    def _prep():
        pb = p_ref[...].astype(jnp.bfloat16)
        pbf_s[...] = pb
        g_s[...] = jax.lax.dot_general(
            pb, pb, (((0,), (0,)), ((), ())),
            preferred_element_type=jnp.float32).astype(jnp.bfloat16)

    @pl.when(i < half)
    def _ingest_step():
        xf = x_ref[...]
        xb = xf.astype(jnp.bfloat16)
        xb_s[pl.ds(i * 2 * _BI, 2 * _BI), :] = xb
        t = jax.lax.dot_general(
            xb, g_s[...], (((1,), (1,)), ((), ())),
            preferred_element_type=jnp.float32)
        s = jnp.sum(t * xf, axis=1, keepdims=True)
        fac_s[pl.ds(i * 2 * _BI, 2 * _BI), :] = 1.0 / (jnp.sqrt(s) + 1e-12)

    xb_i = xb_s[pl.ds(i * _BI, _BI), :]
    xhv_ref[...] = jax.lax.dot_general(
        xb_i, pbf_s[...], (((1,), (1,)), ((), ())),
        preferred_element_type=jnp.float32)

    @pl.when(i >= half)
    def _centroid_step():
        c = c_ref[...]
        cs = jnp.sum(c * c, axis=1, keepdims=True)
        cn = (c * (1.0 / (jnp.sqrt(cs) + 1e-12))).astype(jnp.bfloat16)
        mb = jax.lax.dot_general(
            cn, pbf_s[...], (((1,), (0,)), ((), ())),
            preferred_element_type=jnp.float32).astype(jnp.bfloat16)
        pc = jax.lax.dot_general(
            xb_s[...], mb, (((1,), (1,)), ((), ())),
            preferred_element_type=jnp.float32)
        preds_ref[...] = pc * fac_s[...]


@jax.jit
def kernel(x, projection, centroids):
    B, F = x.shape
    D, _ = projection.shape
    C, _ = centroids.shape
    NS = B // _BI

    xhv, preds = pl.pallas_call(
        _fused_kernel,
        grid=(NS,),
        in_specs=[
            pl.BlockSpec((2 * _BI, F), lambda i: (jnp.minimum(i, 7), 0)),
            pl.BlockSpec((D, F), lambda i: (0, 0)),
            pl.BlockSpec((_BCC, D), lambda i: (jnp.maximum(i - 8, 0), 0)),
        ],
        out_specs=[
            pl.BlockSpec((_BI, D), lambda i: (i, 0)),
            pl.BlockSpec((B, _BCC), lambda i: (0, jnp.maximum(i - 8, 0))),
        ],
        out_shape=[
            jax.ShapeDtypeStruct((B, D), jnp.float32),
            jax.ShapeDtypeStruct((B, C), jnp.float32),
        ],
        scratch_shapes=[
            pltpu.VMEM((B, F), jnp.bfloat16),
            pltpu.VMEM((D, F), jnp.bfloat16),
            pltpu.VMEM((F, F), jnp.bfloat16),
            pltpu.VMEM((B, 1), jnp.float32),
        ],
        compiler_params=pltpu.CompilerParams(
            dimension_semantics=("arbitrary",),
        ),
    )(x, projection, centroids)
    return (preds, xhv)


# speedup vs baseline: 1.0775x; 1.0066x over previous
import jax
import jax.numpy as jnp
from jax.experimental import pallas as pl
from jax.experimental.pallas import tpu as pltpu

_BI = 256
_BCC = 256


def _fused_kernel(x_ref, p_ref, c_ref, xhv_ref, preds_ref,
                  xb_s, pbf_s, g_s, fac_s):
    i = pl.program_id(0)
    half = pl.num_programs(0) // 2

    @pl.when(i == 0)
    def _prep():
        pb = p_ref[...].astype(jnp.bfloat16)
        pbf_s[...] = pb
        g_s[...] = jax.lax.dot_general(
            pb, pb, (((0,), (0,)), ((), ())),
            preferred_element_type=jnp.float32).astype(jnp.bfloat16)

    @pl.when(i < half)
    def _ingest_step():
        xf = x_ref[...]
        xb = xf.astype(jnp.bfloat16)
        xb_s[pl.ds(i * 2 * _BI, 2 * _BI), :] = xb
        t = jax.lax.dot_general(
            xb, g_s[...], (((1,), (1,)), ((), ())),
            preferred_element_type=jnp.float32)
        s = jnp.sum(t * xf, axis=1, keepdims=True)
        fac_s[pl.ds(i * 2 * _BI, 2 * _BI), :] = 1.0 / (jnp.sqrt(s) + 1e-12)

    xb_i = xb_s[pl.ds(i * _BI, _BI), :]
    xhv_ref[...] = jax.lax.dot_general(
        xb_i, pbf_s[...], (((1,), (1,)), ((), ())),
        preferred_element_type=jnp.float32)

    @pl.when((i >= half) & (i < half + 4))
    def _centroid_step():
        c = c_ref[...]
        cs = jnp.sum(c * c, axis=1, keepdims=True)
        cn = (c * (1.0 / (jnp.sqrt(cs) + 1e-12))).astype(jnp.bfloat16)
        mb = jax.lax.dot_general(
            cn, pbf_s[...], (((1,), (0,)), ((), ())),
            preferred_element_type=jnp.float32).astype(jnp.bfloat16)
        pc = jax.lax.dot_general(
            xb_s[...], mb, (((1,), (1,)), ((), ())),
            preferred_element_type=jnp.float32)
        preds_ref[...] = pc * fac_s[...]


@jax.jit
def kernel(x, projection, centroids):
    B, F = x.shape
    D, _ = projection.shape
    C, _ = centroids.shape
    NS = B // _BI

    xhv, preds = pl.pallas_call(
        _fused_kernel,
        grid=(NS,),
        in_specs=[
            pl.BlockSpec((2 * _BI, F), lambda i: (jnp.minimum(i, 7), 0)),
            pl.BlockSpec((D, F), lambda i: (0, 0)),
            pl.BlockSpec((_BCC, D),
                         lambda i: (jnp.clip(i - 8, 0, 3), 0)),
        ],
        out_specs=[
            pl.BlockSpec((_BI, D), lambda i: (i, 0)),
            pl.BlockSpec((B, _BCC),
                         lambda i: (0, jnp.clip(i - 8, 0, 3))),
        ],
        out_shape=[
            jax.ShapeDtypeStruct((B, D), jnp.float32),
            jax.ShapeDtypeStruct((B, C), jnp.float32),
        ],
        scratch_shapes=[
            pltpu.VMEM((B, F), jnp.bfloat16),
            pltpu.VMEM((D, F), jnp.bfloat16),
            pltpu.VMEM((F, F), jnp.bfloat16),
            pltpu.VMEM((B, 1), jnp.float32),
        ],
        compiler_params=pltpu.CompilerParams(
            dimension_semantics=("arbitrary",),
            vmem_limit_bytes=64 * 1024 * 1024,
        ),
    )(x, projection, centroids)
    return (preds, xhv)
